# baseline (speedup 1.0000x reference)
import functools

import jax
import jax.numpy as jnp
from jax.experimental import pallas as pl

_KT = 2000
_NN = 9
_BIG = float("inf")


def _k1_body(emb_ref, mb_ref, min_ref, arg_ref, norm_ref, *, kt):
    i = pl.program_id(0)
    mb = mb_ref[...]
    ynorm = jnp.sum(mb * mb, axis=1)
    norm_ref[0, 0, :] = ynorm
    emb = emb_ref[...]
    xnorm = jnp.sum(emb * emb, axis=1)
    g = jax.lax.dot_general(
        emb, mb, (((1,), (1,)), ((), ())),
        precision=jax.lax.Precision.DEFAULT,
        preferred_element_type=jnp.float32)
    d2 = xnorm[:, None] - 2.0 * g + ynorm[None, :]
    d2 = jnp.maximum(d2, 0.0)
    tmin = jnp.min(d2, axis=1)
    kidx = jax.lax.broadcasted_iota(jnp.int32, d2.shape, 1)
    cand = jnp.where(d2 == tmin[:, None], kidx, kt)
    targ = jnp.min(cand, axis=1) + i * kt

    @pl.when(i == 0)
    def _():
        min_ref[0, :] = tmin
        arg_ref[0, :] = targ

    @pl.when(i > 0)
    def _():
        prev = min_ref[0, :]
        upd = tmin < prev
        min_ref[0, :] = jnp.where(upd, tmin, prev)
        arg_ref[0, :] = jnp.where(upd, targ, arg_ref[0, :])


def _km_body(min_ref, arg_ref, emb_ref, map_ref, feat_ref, nn_ref, sc_ref):
    m2 = min_ref[0, :]
    dist = jnp.sqrt(m2)
    map_ref[0, :] = dist
    n = dist.shape[0]
    s = jnp.max(dist)
    pidx = jax.lax.broadcasted_iota(jnp.int32, (n,), 0)
    p = jnp.min(jnp.where(dist == s, pidx, n))
    nn_idx = jnp.sum(jnp.where(pidx == p, arg_ref[0, :], 0))
    mask = (pidx == p).astype(jnp.float32)[None, :]
    feat = jax.lax.dot_general(
        mask, emb_ref[...], (((1,), (0,)), ((), ())),
        precision=jax.lax.Precision.DEFAULT,
        preferred_element_type=jnp.float32)
    feat_ref[...] = feat
    lane = jax.lax.broadcasted_iota(jnp.int32, nn_ref.shape, 1)
    nn_ref[...] = jnp.where(lane == 0, nn_idx, 0)
    sc_ref[...] = jnp.where(lane == 0, s, 0.0)


def _k2_body(q_ref, mb_ref, norm_ref, val_ref, idx_ref, ev_ref, *, kt):
    i = pl.program_id(0)
    q = q_ref[...]
    qn = jnp.sum(q * q, axis=1)
    mb = mb_ref[...]
    g = jax.lax.dot_general(
        q, mb, (((1,), (1,)), ((), ())),
        precision=jax.lax.Precision.DEFAULT,
        preferred_element_type=jnp.float32)
    d2 = qn[:, None] - 2.0 * g + norm_ref[0, 0, :][None, :]
    d2 = jnp.maximum(d2, 0.0)
    dn = d2[0:1, :]
    de = d2[1:2, :]
    kidx = jax.lax.broadcasted_iota(jnp.int32, (1, kt), 1)
    lane = jax.lax.broadcasted_iota(jnp.int32, (1, 128), 1)
    vals = jnp.full((1, 128), _BIG, jnp.float32)
    idxs = jnp.zeros((1, 128), jnp.int32)
    evs = jnp.zeros((1, 128), jnp.float32)
    d = dn
    for j in range(_NN):
        m = jnp.min(d)
        a = jnp.min(jnp.where(d == m, kidx, kt))
        ev = jnp.sum(jnp.where(kidx == a, de, 0.0))
        vals = jnp.where(lane == j, m, vals)
        idxs = jnp.where(lane == j, a + i * kt, idxs)
        evs = jnp.where(lane == j, ev, evs)
        d = jnp.where(kidx == a, _BIG, d)
    val_ref[...] = vals[None]
    idx_ref[...] = idxs[None]
    ev_ref[...] = evs[None]


def _k3_body(val_ref, idx_ref, ev_ref, sc_ref, pred_ref, *, s):
    v = val_ref[...].reshape(1, s * 128)
    ev = ev_ref[...].reshape(1, s * 128)
    lane = jax.lax.broadcasted_iota(jnp.int32, (1, s * 128), 1)
    lane9 = jax.lax.broadcasted_iota(jnp.int32, (1, 128), 1)
    d3 = jnp.full((1, 128), -_BIG, jnp.float32)
    for j in range(_NN):
        m = jnp.min(v)
        a = jnp.min(jnp.where(v == m, lane, s * 128))
        e = jnp.sum(jnp.where(lane == a, ev, 0.0))
        d3 = jnp.where(lane9 == j, jnp.sqrt(e), d3)
        v = jnp.where(lane == a, _BIG, v)
    valid = lane9 < _NN
    mx = jnp.max(d3)
    num = jnp.where(valid, jnp.exp(d3 - mx), 0.0)
    w0 = jnp.sum(jnp.where(lane9 == 0, num, 0.0)) / jnp.sum(num)
    score = sc_ref[0, 0]
    out_lane = jax.lax.broadcasted_iota(jnp.int32, pred_ref.shape, 1)
    pred_ref[...] = jnp.where(out_lane == 0, (1.0 - w0) * score, 0.0)


@jax.jit
def kernel(embedding, memory_bank):
    n, d = embedding.shape
    k = memory_bank.shape[0]
    kt = _KT
    steps = k // kt

    min2, argm, norms = pl.pallas_call(
        functools.partial(_k1_body, kt=kt),
        grid=(steps,),
        in_specs=[
            pl.BlockSpec((n, d), lambda i: (0, 0)),
            pl.BlockSpec((kt, d), lambda i: (i, 0)),
        ],
        out_specs=[
            pl.BlockSpec((1, n), lambda i: (0, 0)),
            pl.BlockSpec((1, n), lambda i: (0, 0)),
            pl.BlockSpec((1, 1, kt), lambda i: (i, 0, 0)),
        ],
        out_shape=[
            jax.ShapeDtypeStruct((1, n), jnp.float32),
            jax.ShapeDtypeStruct((1, n), jnp.int32),
            jax.ShapeDtypeStruct((steps, 1, kt), jnp.float32),
        ],
    )(embedding, memory_bank)

    amap, feat, nn_idx, scorev = pl.pallas_call(
        _km_body,
        out_shape=[
            jax.ShapeDtypeStruct((1, n), jnp.float32),
            jax.ShapeDtypeStruct((1, d), jnp.float32),
            jax.ShapeDtypeStruct((1, 128), jnp.int32),
            jax.ShapeDtypeStruct((1, 128), jnp.float32),
        ],
    )(min2, argm, embedding)

    nn_sample = memory_bank[nn_idx[0, 0]]
    q = jnp.concatenate([nn_sample[None, :], feat], axis=0)

    vals, idxs, evs = pl.pallas_call(
        functools.partial(_k2_body, kt=kt),
        grid=(steps,),
        in_specs=[
            pl.BlockSpec((2, d), lambda i: (0, 0)),
            pl.BlockSpec((kt, d), lambda i: (i, 0)),
            pl.BlockSpec((1, 1, kt), lambda i: (i, 0, 0)),
        ],
        out_specs=[
            pl.BlockSpec((1, 1, 128), lambda i: (i, 0, 0)),
            pl.BlockSpec((1, 1, 128), lambda i: (i, 0, 0)),
            pl.BlockSpec((1, 1, 128), lambda i: (i, 0, 0)),
        ],
        out_shape=[
            jax.ShapeDtypeStruct((steps, 1, 128), jnp.float32),
            jax.ShapeDtypeStruct((steps, 1, 128), jnp.int32),
            jax.ShapeDtypeStruct((steps, 1, 128), jnp.float32),
        ],
    )(q, memory_bank, norms)

    pred = pl.pallas_call(
        functools.partial(_k3_body, s=steps),
        out_shape=jax.ShapeDtypeStruct((1, 128), jnp.float32),
    )(vals, idxs, evs, scorev)

    pred_score = pred[0, 0:1]
    anomaly_map = amap.reshape(1, 1, 28, 28)
    locations = argm.reshape(1, n)
    return pred_score, anomaly_map, locations


# speedup vs baseline: 1.4409x; 1.4409x over previous
import functools

import jax
import jax.numpy as jnp
from jax.experimental import pallas as pl

_KT = 2000
_NN = 9
_BIG = float("inf")


def _k1_body(emb_ref, mb_ref, min_ref, arg_ref, norm_ref, *, kt):
    i = pl.program_id(0)
    mb = mb_ref[...]
    ynorm = jnp.sum(mb * mb, axis=1)
    norm_ref[0, 0, :] = ynorm
    emb = emb_ref[...]
    xnorm = jnp.sum(emb * emb, axis=1)
    g = jax.lax.dot_general(
        emb, mb, (((1,), (1,)), ((), ())),
        precision=jax.lax.Precision.DEFAULT,
        preferred_element_type=jnp.float32)
    d2 = xnorm[:, None] - 2.0 * g + ynorm[None, :]
    tmin = jnp.min(d2, axis=1)
    targ = jnp.argmin(d2, axis=1).astype(jnp.int32) + i * kt

    @pl.when(i == 0)
    def _():
        min_ref[0, :] = tmin
        arg_ref[0, :] = targ

    @pl.when(i > 0)
    def _():
        prev = min_ref[0, :]
        upd = tmin < prev
        min_ref[0, :] = jnp.where(upd, tmin, prev)
        arg_ref[0, :] = jnp.where(upd, targ, arg_ref[0, :])


def _km_body(min_ref, arg_ref, emb_ref, map_ref, feat_ref, nn_ref, sc_ref):
    m2 = min_ref[0, :]
    dist = jnp.sqrt(jnp.maximum(m2, 0.0))
    map_ref[0, :] = dist
    n = dist.shape[0]
    s = jnp.max(dist)
    pidx = jax.lax.broadcasted_iota(jnp.int32, (n,), 0)
    p = jnp.min(jnp.where(dist == s, pidx, n))
    nn_idx = jnp.sum(jnp.where(pidx == p, arg_ref[0, :], 0))
    mask = (pidx == p).astype(jnp.float32)[None, :]
    feat = jax.lax.dot_general(
        mask, emb_ref[...], (((1,), (0,)), ((), ())),
        precision=jax.lax.Precision.DEFAULT,
        preferred_element_type=jnp.float32)
    feat_ref[...] = feat
    lane = jax.lax.broadcasted_iota(jnp.int32, nn_ref.shape, 1)
    nn_ref[...] = jnp.where(lane == 0, nn_idx, 0)
    sc_ref[...] = jnp.where(lane == 0, s, 0.0)


def _k2_body(q_ref, mb_ref, norm_ref, dn_ref, de_ref):
    q = q_ref[...]
    qn = jnp.sum(q * q, axis=1)
    mb = mb_ref[...]
    g = jax.lax.dot_general(
        q, mb, (((1,), (1,)), ((), ())),
        precision=jax.lax.Precision.DEFAULT,
        preferred_element_type=jnp.float32)
    d2 = qn[:, None] - 2.0 * g + norm_ref[0, 0, :][None, :]
    d2 = jnp.maximum(d2, 0.0)
    dn_ref[0, ...] = d2[0:1, :]
    de_ref[0, ...] = d2[1:2, :]


def _k3_body(dn_ref, de_ref, sc_ref, pred_ref, *, s, kt):
    v = dn_ref[...].reshape(s, kt)
    ev = de_ref[...].reshape(s, kt)
    row = jax.lax.broadcasted_iota(jnp.int32, (s, kt), 0)
    col = jax.lax.broadcasted_iota(jnp.int32, (s, kt), 1)
    kidx = row * kt + col
    lane9 = jax.lax.broadcasted_iota(jnp.int32, (1, 128), 1)
    d3 = jnp.full((1, 128), -_BIG, jnp.float32)
    for j in range(_NN):
        m = jnp.min(v)
        a = jnp.min(jnp.where(v == m, kidx, s * kt))
        sel = kidx == a
        e = jnp.sum(jnp.where(sel, ev, 0.0))
        d3 = jnp.where(lane9 == j, jnp.sqrt(e), d3)
        v = jnp.where(sel, _BIG, v)
    valid = lane9 < _NN
    mx = jnp.max(d3)
    num = jnp.where(valid, jnp.exp(d3 - mx), 0.0)
    w0 = jnp.sum(jnp.where(lane9 == 0, num, 0.0)) / jnp.sum(num)
    score = sc_ref[0, 0]
    out_lane = jax.lax.broadcasted_iota(jnp.int32, pred_ref.shape, 1)
    pred_ref[...] = jnp.where(out_lane == 0, (1.0 - w0) * score, 0.0)


@jax.jit
def kernel(embedding, memory_bank):
    n, d = embedding.shape
    k = memory_bank.shape[0]
    kt = _KT
    steps = k // kt

    min2, argm, norms = pl.pallas_call(
        functools.partial(_k1_body, kt=kt),
        grid=(steps,),
        in_specs=[
            pl.BlockSpec((n, d), lambda i: (0, 0)),
            pl.BlockSpec((kt, d), lambda i: (i, 0)),
        ],
        out_specs=[
            pl.BlockSpec((1, n), lambda i: (0, 0)),
            pl.BlockSpec((1, n), lambda i: (0, 0)),
            pl.BlockSpec((1, 1, kt), lambda i: (i, 0, 0)),
        ],
        out_shape=[
            jax.ShapeDtypeStruct((1, n), jnp.float32),
            jax.ShapeDtypeStruct((1, n), jnp.int32),
            jax.ShapeDtypeStruct((steps, 1, kt), jnp.float32),
        ],
    )(embedding, memory_bank)

    amap, feat, nn_idx, scorev = pl.pallas_call(
        _km_body,
        out_shape=[
            jax.ShapeDtypeStruct((1, n), jnp.float32),
            jax.ShapeDtypeStruct((1, d), jnp.float32),
            jax.ShapeDtypeStruct((1, 128), jnp.int32),
            jax.ShapeDtypeStruct((1, 128), jnp.float32),
        ],
    )(min2, argm, embedding)

    nn_sample = memory_bank[nn_idx[0, 0]]
    q = jnp.concatenate([nn_sample[None, :], feat], axis=0)

    dn, de = pl.pallas_call(
        _k2_body,
        grid=(steps,),
        in_specs=[
            pl.BlockSpec((2, d), lambda i: (0, 0)),
            pl.BlockSpec((kt, d), lambda i: (i, 0)),
            pl.BlockSpec((1, 1, kt), lambda i: (i, 0, 0)),
        ],
        out_specs=[
            pl.BlockSpec((1, 1, kt), lambda i: (i, 0, 0)),
            pl.BlockSpec((1, 1, kt), lambda i: (i, 0, 0)),
        ],
        out_shape=[
            jax.ShapeDtypeStruct((steps, 1, kt), jnp.float32),
            jax.ShapeDtypeStruct((steps, 1, kt), jnp.float32),
        ],
    )(q, memory_bank, norms)

    pred = pl.pallas_call(
        functools.partial(_k3_body, s=steps, kt=kt),
        out_shape=jax.ShapeDtypeStruct((1, 128), jnp.float32),
    )(dn, de, scorev)

    pred_score = pred[0, 0:1]
    anomaly_map = amap.reshape(1, 1, 28, 28)
    locations = argm.reshape(1, n)
    return pred_score, anomaly_map, locations
